# Initial kernel scaffold; baseline (speedup 1.0000x reference)
import jax
import jax.numpy as jnp
from jax.experimental import pallas as pl


def kernel(predictions, targets, penalty_matrix):
    raise NotImplementedError("write your pallas kernel here")


# speedup vs baseline: 1057.4871x; 1057.4871x over previous
import functools

import jax
import jax.numpy as jnp
from jax import lax
from jax.experimental import pallas as pl
from jax.experimental.pallas import tpu as pltpu
from jax.experimental.pallas import tpu_sc as plsc

_N = 8388608
_MSE_WEIGHT = 0.3
_NC = 2
_NS = 16
_NW = _NC * _NS
_E = _N // _NW
_C = 32768
_LANES = 16


def _body(num_classes, pred_hbm, tgt_hbm, pmat_hbm, out_pen_hbm, out_mse_hbm,
          pbuf, tbuf, pmat_v, accv_pen, accv_mse):
    wid = lax.axis_index("c") * _NS + lax.axis_index("s")
    base = wid * _E

    pltpu.sync_copy(pmat_hbm, pmat_v)

    top = jnp.float32(num_classes - 1)

    def vreg_step(j, carry):
        acc_pen, acc_mse = carry
        p = pbuf[pl.ds(j * _LANES, _LANES)]
        t = tbuf[pl.ds(j * _LANES, _LANES)]
        cp = jnp.minimum(jnp.maximum(p, 0.0), top)
        li = cp.astype(jnp.int32)
        wu = cp - li.astype(jnp.float32)
        idx_lo = t * num_classes + li
        idx_hi = jnp.where(wu > 0.0, idx_lo + 1, idx_lo)
        pen_lo = plsc.load_gather(pmat_v, [idx_lo])
        pen_hi = plsc.load_gather(pmat_v, [idx_hi])
        interp = pen_lo + wu * (pen_hi - pen_lo)
        d = p - t.astype(jnp.float32)
        return acc_pen + interp, acc_mse + d * d

    acc_pen = jnp.zeros((_LANES,), jnp.float32)
    acc_mse = jnp.zeros((_LANES,), jnp.float32)
    for k in range(_E // _C):
        pltpu.sync_copy(pred_hbm.at[pl.ds(base + k * _C, _C)], pbuf)
        pltpu.sync_copy(tgt_hbm.at[pl.ds(base + k * _C, _C)], tbuf)
        acc_pen, acc_mse = lax.fori_loop(
            0, _C // _LANES, vreg_step, (acc_pen, acc_mse))

    accv_pen[...] = acc_pen
    accv_mse[...] = acc_mse
    pltpu.sync_copy(accv_pen, out_pen_hbm.at[wid])
    pltpu.sync_copy(accv_mse, out_mse_hbm.at[wid])


def kernel(predictions, targets, penalty_matrix):
    num_classes = penalty_matrix.shape[0]
    pmat_flat = penalty_matrix.reshape(-1)

    mesh = plsc.VectorSubcoreMesh(core_axis_name="c", subcore_axis_name="s")
    run = pl.kernel(
        functools.partial(_body, num_classes),
        out_type=[
            jax.ShapeDtypeStruct((_NW, _LANES), jnp.float32),
            jax.ShapeDtypeStruct((_NW, _LANES), jnp.float32),
        ],
        mesh=mesh,
        scratch_types=[
            pltpu.VMEM((_C,), jnp.float32),
            pltpu.VMEM((_C,), jnp.int32),
            pltpu.VMEM((num_classes * num_classes,), jnp.float32),
            pltpu.VMEM((_LANES,), jnp.float32),
            pltpu.VMEM((_LANES,), jnp.float32),
        ],
        compiler_params=pltpu.CompilerParams(needs_layout_passes=False),
        name="soft_penalty_loss_sc",
    )
    pen_part, mse_part = run(predictions, targets, pmat_flat)
    inv_n = jnp.float32(1.0 / _N)
    return pen_part.sum() * inv_n + _MSE_WEIGHT * (mse_part.sum() * inv_n)


# speedup vs baseline: 1403.7421x; 1.3274x over previous
import functools

import jax
import jax.numpy as jnp
from jax import lax
from jax.experimental import pallas as pl
from jax.experimental.pallas import tpu as pltpu
from jax.experimental.pallas import tpu_sc as plsc

_N = 8388608
_MSE_WEIGHT = 0.3
_NC = 2
_NS = 16
_NW = _NC * _NS
_E = _N // _NW
_C = 16384
_NCH = _E // _C
_LANES = 16


def _body(num_classes, pred_hbm, tgt_hbm, pmat_hbm, dmat_hbm,
          out_pen_hbm, out_mse_hbm,
          pbuf0, pbuf1, tbuf0, tbuf1, pmat_v, dmat_v, accv_pen, accv_mse,
          sem_p0, sem_p1, sem_t0, sem_t1):
    wid = lax.axis_index("c") * _NS + lax.axis_index("s")
    base = wid * _E

    pltpu.sync_copy(pmat_hbm, pmat_v)
    pltpu.sync_copy(dmat_hbm, dmat_v)

    top = jnp.float32(num_classes - 1)
    pbufs = (pbuf0, pbuf1)
    tbufs = (tbuf0, tbuf1)
    psem = (sem_p0, sem_p1)
    tsem = (sem_t0, sem_t1)

    def start(k):
        b = k % 2
        return (
            pltpu.async_copy(pred_hbm.at[pl.ds(base + k * _C, _C)],
                             pbufs[b], psem[b]),
            pltpu.async_copy(tgt_hbm.at[pl.ds(base + k * _C, _C)],
                             tbufs[b], tsem[b]),
        )

    def make_step(pb, tb):
        def vreg_step(j, carry):
            acc_pen, acc_mse = carry
            p = pb[pl.ds(j * _LANES, _LANES)]
            t = tb[pl.ds(j * _LANES, _LANES)]
            cp = jnp.minimum(jnp.maximum(p, 0.0), top)
            li = cp.astype(jnp.int32)
            wu = cp - li.astype(jnp.float32)
            idx = t * num_classes + li
            pen = plsc.load_gather(pmat_v, [idx])
            dlt = plsc.load_gather(dmat_v, [idx])
            d = p - t.astype(jnp.float32)
            return acc_pen + (pen + wu * dlt), acc_mse + d * d
        return vreg_step

    acc_pen = jnp.zeros((_LANES,), jnp.float32)
    acc_mse = jnp.zeros((_LANES,), jnp.float32)
    pending = start(0)
    for k in range(_NCH):
        b = k % 2
        for cp_desc in pending:
            cp_desc.wait()
        if k + 1 < _NCH:
            pending = start(k + 1)
        acc_pen, acc_mse = lax.fori_loop(
            0, _C // _LANES, make_step(pbufs[b], tbufs[b]),
            (acc_pen, acc_mse))

    accv_pen[...] = acc_pen
    accv_mse[...] = acc_mse
    pltpu.sync_copy(accv_pen, out_pen_hbm.at[wid])
    pltpu.sync_copy(accv_mse, out_mse_hbm.at[wid])


def kernel(predictions, targets, penalty_matrix):
    num_classes = penalty_matrix.shape[0]
    pmat_flat = penalty_matrix.reshape(-1)
    dmat_flat = jnp.concatenate(
        [penalty_matrix[:, 1:] - penalty_matrix[:, :-1],
         jnp.zeros((num_classes, 1), penalty_matrix.dtype)],
        axis=1).reshape(-1)

    mesh = plsc.VectorSubcoreMesh(core_axis_name="c", subcore_axis_name="s")
    run = pl.kernel(
        functools.partial(_body, num_classes),
        out_type=[
            jax.ShapeDtypeStruct((_NW, _LANES), jnp.float32),
            jax.ShapeDtypeStruct((_NW, _LANES), jnp.float32),
        ],
        mesh=mesh,
        scratch_types=[
            pltpu.VMEM((_C,), jnp.float32),
            pltpu.VMEM((_C,), jnp.float32),
            pltpu.VMEM((_C,), jnp.int32),
            pltpu.VMEM((_C,), jnp.int32),
            pltpu.VMEM((num_classes * num_classes,), jnp.float32),
            pltpu.VMEM((num_classes * num_classes,), jnp.float32),
            pltpu.VMEM((_LANES,), jnp.float32),
            pltpu.VMEM((_LANES,), jnp.float32),
            pltpu.SemaphoreType.DMA,
            pltpu.SemaphoreType.DMA,
            pltpu.SemaphoreType.DMA,
            pltpu.SemaphoreType.DMA,
        ],
        compiler_params=pltpu.CompilerParams(needs_layout_passes=False),
        name="soft_penalty_loss_sc",
    )
    pen_part, mse_part = run(predictions, targets, pmat_flat, dmat_flat)
    inv_n = jnp.float32(1.0 / _N)
    return pen_part.sum() * inv_n + _MSE_WEIGHT * (mse_part.sum() * inv_n)


# speedup vs baseline: 1547.4350x; 1.1024x over previous
import functools

import jax
import jax.numpy as jnp
from jax import lax
from jax.experimental import pallas as pl
from jax.experimental.pallas import tpu as pltpu
from jax.experimental.pallas import tpu_sc as plsc

_N = 8388608
_MSE_WEIGHT = 0.3
_NC = 2
_NS = 16
_NW = _NC * _NS
_E = _N // _NW
_C = 16384
_NCH = _E // _C
_LANES = 16


def _body(num_classes, pred_hbm, tgt_hbm, pmat_hbm, dmat_hbm,
          out_pen_hbm, out_mse_hbm,
          pbuf0, pbuf1, tbuf0, tbuf1, pmat_v, dmat_v, accv_pen, accv_mse,
          sem_p0, sem_p1, sem_t0, sem_t1):
    wid = lax.axis_index("c") * _NS + lax.axis_index("s")
    base = wid * _E

    pltpu.sync_copy(pmat_hbm, pmat_v)
    pltpu.sync_copy(dmat_hbm, dmat_v)

    top = jnp.float32(num_classes - 1)
    pbufs = (pbuf0, pbuf1)
    tbufs = (tbuf0, tbuf1)
    psem = (sem_p0, sem_p1)
    tsem = (sem_t0, sem_t1)

    def start(k):
        b = k % 2
        return (
            pltpu.async_copy(pred_hbm.at[pl.ds(base + k * _C, _C)],
                             pbufs[b], psem[b]),
            pltpu.async_copy(tgt_hbm.at[pl.ds(base + k * _C, _C)],
                             tbufs[b], tsem[b]),
        )

    def chunk_sum(pb, tb, acc):
        @plsc.parallel_loop(0, _C // _LANES, carry=acc, unroll=8)
        def result(j, carry):
            acc_pen, acc_mse = carry
            p = pb[pl.ds(j * _LANES, _LANES)]
            t = tb[pl.ds(j * _LANES, _LANES)]
            cp = jnp.minimum(jnp.maximum(p, 0.0), top)
            li = cp.astype(jnp.int32)
            wu = cp - li.astype(jnp.float32)
            idx = t * num_classes + li
            pen = plsc.load_gather(pmat_v, [idx])
            dlt = plsc.load_gather(dmat_v, [idx])
            d = p - t.astype(jnp.float32)
            return acc_pen + (pen + wu * dlt), acc_mse + d * d
        return result

    acc = (jnp.zeros((_LANES,), jnp.float32),
           jnp.zeros((_LANES,), jnp.float32))
    pending = start(0)
    for k in range(_NCH):
        b = k % 2
        for cp_desc in pending:
            cp_desc.wait()
        if k + 1 < _NCH:
            pending = start(k + 1)
        acc = chunk_sum(pbufs[b], tbufs[b], acc)
    acc_pen, acc_mse = acc

    accv_pen[...] = acc_pen
    accv_mse[...] = acc_mse
    pltpu.sync_copy(accv_pen, out_pen_hbm.at[wid])
    pltpu.sync_copy(accv_mse, out_mse_hbm.at[wid])


def kernel(predictions, targets, penalty_matrix):
    num_classes = penalty_matrix.shape[0]
    pmat_flat = penalty_matrix.reshape(-1)
    dmat_flat = jnp.concatenate(
        [penalty_matrix[:, 1:] - penalty_matrix[:, :-1],
         jnp.zeros((num_classes, 1), penalty_matrix.dtype)],
        axis=1).reshape(-1)

    mesh = plsc.VectorSubcoreMesh(core_axis_name="c", subcore_axis_name="s")
    run = pl.kernel(
        functools.partial(_body, num_classes),
        out_type=[
            jax.ShapeDtypeStruct((_NW, _LANES), jnp.float32),
            jax.ShapeDtypeStruct((_NW, _LANES), jnp.float32),
        ],
        mesh=mesh,
        scratch_types=[
            pltpu.VMEM((_C,), jnp.float32),
            pltpu.VMEM((_C,), jnp.float32),
            pltpu.VMEM((_C,), jnp.int32),
            pltpu.VMEM((_C,), jnp.int32),
            pltpu.VMEM((num_classes * num_classes,), jnp.float32),
            pltpu.VMEM((num_classes * num_classes,), jnp.float32),
            pltpu.VMEM((_LANES,), jnp.float32),
            pltpu.VMEM((_LANES,), jnp.float32),
            pltpu.SemaphoreType.DMA,
            pltpu.SemaphoreType.DMA,
            pltpu.SemaphoreType.DMA,
            pltpu.SemaphoreType.DMA,
        ],
        compiler_params=pltpu.CompilerParams(needs_layout_passes=False),
        name="soft_penalty_loss_sc",
    )
    pen_part, mse_part = run(predictions, targets, pmat_flat, dmat_flat)
    inv_n = jnp.float32(1.0 / _N)
    return pen_part.sum() * inv_n + _MSE_WEIGHT * (mse_part.sum() * inv_n)
